# baseline (speedup 1.0000x reference)
import functools

import jax
import jax.numpy as jnp
from jax import lax
from jax.experimental import pallas as pl
from jax.experimental.pallas import tpu as pltpu
from jax.experimental.pallas import tpu_sc as plsc

VOCAB = 1_000_000
EMBED = 64
N = 4096 * 200

NC = 2
NS = 16
NW = NC * NS
BPW = N // NW

DMA_ROWS = 128
CHUNK = 512
DPC = CHUNK // DMA_ROWS
NCHUNK = BPW // CHUNK
IPW = BPW // DMA_ROWS


def _sc_gather_fn():
    mesh = plsc.VectorSubcoreMesh(
        core_axis_name="c", subcore_axis_name="s", num_cores=NC, num_subcores=NS
    )

    @functools.partial(
        pl.kernel,
        mesh=mesh,
        compiler_params=pltpu.CompilerParams(use_tc_tiling_on_sc=False),
        out_type=jax.ShapeDtypeStruct((N // DMA_ROWS, DMA_ROWS, EMBED), jnp.float32),
        scratch_types=[
            pltpu.VMEM((IPW, DMA_ROWS), jnp.int32),
            pltpu.VMEM((DPC, DMA_ROWS, EMBED), jnp.float32),
            pltpu.SemaphoreType.DMA,
        ],
    )
    def sc_gather(table_hbm, idx_hbm, out_hbm, idx_v, rows_v, gsem):
        cid = lax.axis_index("c")
        sid = lax.axis_index("s")
        wid = sid * NC + cid
        base = wid * (BPW // DMA_ROWS)

        pltpu.sync_copy(idx_hbm.at[wid], idx_v)

        @pl.loop(0, NCHUNK)
        def chunk_loop(g):
            copies = []
            for kk in range(DPC):
                j = g * DPC + kk
                cp = pltpu.async_copy(
                    table_hbm.at[idx_v.at[j]],
                    rows_v.at[kk],
                    gsem,
                )
                copies.append(cp)
            for cp in copies:
                cp.wait()
            pltpu.sync_copy(rows_v, out_hbm.at[pl.ds(base + g * DPC, DPC)])

    return sc_gather


_PB = 2048
_GRID = (N // 2) // _PB


def _tc_loss_body(x2_ref, te_ref, to_ref, logits_ref, out_ref):
    x2 = x2_ref[...]
    xl = x2[:, 0:EMBED]
    xr = x2[:, EMBED:2 * EMBED]
    xl3 = xl.reshape(_PB // 128, 128, EMBED)
    xr3 = xr.reshape(_PB // 128, 128, EMBED)

    def half_loss(x3, t):
        m = jnp.max(x3, axis=2, keepdims=True)
        s = jnp.sum(jnp.exp(x3 - m), axis=2, keepdims=True)
        logz = m + jnp.log(s)
        onehot = lax.broadcasted_iota(jnp.int32, x3.shape, 2) == t[:, :, None]
        picked = jnp.sum(jnp.where(onehot, x3, 0.0), axis=2)
        return jnp.sum(logz) - jnp.sum(picked)

    part = (half_loss(xl3, te_ref[...]) + half_loss(xr3, to_ref[...])) * (1.0 / N)
    part = jnp.reshape(part, (1, 1))

    y = jnp.stack([xl3, xr3], axis=2)
    logits_ref[...] = y.reshape(_PB // 64, 128, EMBED)

    @pl.when(pl.program_id(0) == 0)
    def _init():
        out_ref[...] = jnp.zeros_like(out_ref)

    out_ref[...] += part


def _tc_loss(x2, te2, to2):
    return pl.pallas_call(
        _tc_loss_body,
        grid=(_GRID,),
        in_specs=[
            pl.BlockSpec((_PB, 128), lambda i: (i, 0)),
            pl.BlockSpec((_PB // 128, 128), lambda i: (i, 0)),
            pl.BlockSpec((_PB // 128, 128), lambda i: (i, 0)),
        ],
        out_specs=[
            pl.BlockSpec((_PB // 64, 128, EMBED), lambda i: (i, 0, 0)),
            pl.BlockSpec((1, 1), lambda i: (0, 0)),
        ],
        out_shape=[
            jax.ShapeDtypeStruct((N // 128, 128, EMBED), jnp.float32),
            jax.ShapeDtypeStruct((1, 1), jnp.float32),
        ],
    )(x2, te2, to2)


def kernel(idx, targets, table):
    idx3 = idx.astype(jnp.int32).reshape(NW, IPW, DMA_ROWS)
    gathered = _sc_gather_fn()(table, idx3)
    x2 = gathered.reshape(N // 2, 2 * EMBED)
    t2 = targets.astype(jnp.int32).reshape(N // 2, 2)
    te2 = t2[:, 0].reshape(N // 256, 128)
    to2 = t2[:, 1].reshape(N // 256, 128)
    logits3, loss = _tc_loss(x2, te2, to2)
    return (logits3.reshape(N, EMBED), loss[0, 0])


# speedup vs baseline: 1.3792x; 1.3792x over previous
import functools

import jax
import jax.numpy as jnp
from jax import lax
from jax.experimental import pallas as pl
from jax.experimental.pallas import tpu as pltpu
from jax.experimental.pallas import tpu_sc as plsc

VOCAB = 1_000_000
EMBED = 64
N = 4096 * 200

NC = 2
NS = 16
NW = NC * NS
BPW = N // NW

DMA_ROWS = 128
CHUNK = 512
DPC = CHUNK // DMA_ROWS
NCHUNK = BPW // CHUNK
IPW = BPW // DMA_ROWS


def _sc_gather_fn():
    mesh = plsc.VectorSubcoreMesh(
        core_axis_name="c", subcore_axis_name="s", num_cores=NC, num_subcores=NS
    )

    @functools.partial(
        pl.kernel,
        mesh=mesh,
        compiler_params=pltpu.CompilerParams(use_tc_tiling_on_sc=False),
        out_type=jax.ShapeDtypeStruct((N, EMBED), jnp.float32),
        scratch_types=[
            pltpu.VMEM((IPW, DMA_ROWS), jnp.int32),
            pltpu.VMEM((CHUNK, EMBED), jnp.float32),
            pltpu.SemaphoreType.DMA,
        ],
    )
    def sc_gather(table_hbm, idx_hbm, out_hbm, idx_v, rows_v, gsem):
        cid = lax.axis_index("c")
        sid = lax.axis_index("s")
        wid = sid * NC + cid
        base = wid * BPW

        pltpu.sync_copy(idx_hbm.at[wid], idx_v)

        @pl.loop(0, NCHUNK)
        def chunk_loop(g):
            copies = []
            for kk in range(DPC):
                j = g * DPC + kk
                cp = pltpu.async_copy(
                    table_hbm.at[idx_v.at[j]],
                    rows_v.at[pl.ds(kk * DMA_ROWS, DMA_ROWS)],
                    gsem,
                )
                copies.append(cp)
            for cp in copies:
                cp.wait()
            pltpu.sync_copy(rows_v, out_hbm.at[pl.ds(base + g * CHUNK, CHUNK)])

    return sc_gather


_RB = 4096
_GRID = N // _RB


def _tc_loss_body(x_ref, t_ref, out_ref):
    x = x_ref[...]
    x3 = x.reshape(_RB // 128, 128, EMBED)
    t = t_ref[...]
    m = jnp.max(x3, axis=2, keepdims=True)
    s = jnp.sum(jnp.exp(x3 - m), axis=2, keepdims=True)
    logz = m + jnp.log(s)
    onehot = lax.broadcasted_iota(jnp.int32, x3.shape, 2) == t[:, :, None]
    picked = jnp.sum(jnp.where(onehot, x3, 0.0), axis=2)
    part = (jnp.sum(logz) - jnp.sum(picked)) * (1.0 / N)
    part = jnp.reshape(part, (1, 1))

    @pl.when(pl.program_id(0) == 0)
    def _init():
        out_ref[...] = jnp.zeros_like(out_ref)

    out_ref[...] += part


def _tc_loss(logits, tgt2):
    return pl.pallas_call(
        _tc_loss_body,
        grid=(_GRID,),
        in_specs=[
            pl.BlockSpec((_RB, EMBED), lambda i: (i, 0)),
            pl.BlockSpec((_RB // 128, 128), lambda i: (i, 0)),
        ],
        out_specs=pl.BlockSpec((1, 1), lambda i: (0, 0)),
        out_shape=jax.ShapeDtypeStruct((1, 1), jnp.float32),
    )(logits, tgt2)


def kernel(idx, targets, table):
    idx3 = idx.astype(jnp.int32).reshape(NW, IPW, DMA_ROWS)
    logits = _sc_gather_fn()(table, idx3)
    loss = _tc_loss(logits, targets.astype(jnp.int32).reshape(N // 128, 128))
    return (logits, loss[0, 0])
